# baseline (speedup 1.0000x reference)
import functools

import jax
import jax.numpy as jnp
from jax import lax
from jax.experimental import pallas as pl
from jax.experimental.pallas import tpu as pltpu
from jax.experimental.pallas import tpu_sc as plsc

_NC = 2
_NS = 16
_NW = _NC * _NS
_L = 16

_SC_ROWS = 2048


def _mask_body(x_ref, o_ref):
    v = x_ref[...]
    o_ref[...] = jnp.where(v > 0, v, 0.0)


def kernel(x):
    n_rows, n_cols = x.shape
    tc_rows = n_rows - _SC_ROWS
    sc_elems = _SC_ROWS * n_cols
    per_w = sc_elems // _NW
    sc_base = tc_rows * n_cols

    blk = tc_rows // 2
    tc_out = pl.pallas_call(
        _mask_body,
        out_shape=jax.ShapeDtypeStruct((tc_rows, n_cols), jnp.float32),
        grid=(2,),
        in_specs=[pl.BlockSpec((blk, n_cols), lambda i: (i, 0))],
        out_specs=pl.BlockSpec((blk, n_cols), lambda i: (i, 0)),
    )(x)

    mesh = plsc.VectorSubcoreMesh(core_axis_name="c", subcore_axis_name="s")

    @functools.partial(
        pl.kernel,
        mesh=mesh,
        out_type=jax.ShapeDtypeStruct((sc_elems,), jnp.float32),
        scratch_types=[pltpu.VMEM((per_w,), jnp.float32)],
    )
    def sc_mask(x_hbm, out_hbm, buf):
        wid = lax.axis_index("s") * _NC + lax.axis_index("c")
        base = wid * per_w
        pltpu.sync_copy(x_hbm.at[pl.ds(sc_base + base, per_w)], buf)

        @plsc.parallel_loop(0, per_w, step=_L, unroll=8)
        def _(i):
            v = buf[pl.ds(i, _L)]
            buf[pl.ds(i, _L)] = jnp.where(v > 0, v, 0.0)

        pltpu.sync_copy(buf, out_hbm.at[pl.ds(base, per_w)])

    sc_out = sc_mask(x.reshape(n_rows * n_cols))
    return jnp.concatenate(
        [tc_out, sc_out.reshape(_SC_ROWS, n_cols)], axis=0
    )


# speedup vs baseline: 3.9304x; 3.9304x over previous
import jax
import jax.numpy as jnp
from jax.experimental import pallas as pl


_BLK_ROWS = 16384


def _mask_kernel(x_ref, o_ref):
    x = x_ref[...]
    o_ref[...] = jnp.where(x > 0, x, 0.0)


def kernel(x):
    n_rows, n_cols = x.shape
    grid = (n_rows // _BLK_ROWS,)
    return pl.pallas_call(
        _mask_kernel,
        out_shape=jax.ShapeDtypeStruct(x.shape, x.dtype),
        grid=grid,
        in_specs=[pl.BlockSpec((_BLK_ROWS, n_cols), lambda i: (i, 0))],
        out_specs=pl.BlockSpec((_BLK_ROWS, n_cols), lambda i: (i, 0)),
    )(x)


# speedup vs baseline: 4.8596x; 1.2364x over previous
import jax
import jax.numpy as jnp
from jax.experimental import pallas as pl


_BLK_ROWS = 8192


def _mask_kernel(x_ref, o_ref):
    x = x_ref[...]
    o_ref[...] = jnp.where(x > 0, x, 0.0)


def kernel(x):
    n_rows, n_cols = x.shape
    grid = (n_rows // _BLK_ROWS,)
    return pl.pallas_call(
        _mask_kernel,
        out_shape=jax.ShapeDtypeStruct(x.shape, x.dtype),
        grid=grid,
        in_specs=[pl.BlockSpec((_BLK_ROWS, n_cols), lambda i: (i, 0))],
        out_specs=pl.BlockSpec((_BLK_ROWS, n_cols), lambda i: (i, 0)),
    )(x)
